# Initial kernel scaffold; baseline (speedup 1.0000x reference)
import jax
import jax.numpy as jnp
from jax.experimental import pallas as pl


def kernel(x, coeff):
    raise NotImplementedError("write your pallas kernel here")


# speedup vs baseline: 61.1807x; 61.1807x over previous
import functools

import jax
import jax.numpy as jnp
import numpy as np
from jax import lax
from jax.experimental import pallas as pl
from jax.experimental.pallas import tpu as pltpu
from jax.experimental.pallas import tpu_sc as plsc

XMIN = 0.0
XMAX = 1.0

NC = 2
NS = 16
L = 16
NW = NC * NS

C = 2048
GROUP = 128
G = C // GROUP
JJ = C // L


def _spline_body(n, pt, x_hbm, c4_hbm, y_hbm,
                 x_v, i0_v, i1_v, i2_v, i3_v,
                 w0_v, w1_v, w2_v, w3_v,
                 c0_v, c1_v, c2_v, c3_v, y_v, sem):
    h = (XMAX - XMIN) / (n - 1)
    eps = float(np.finfo(np.float32).eps)
    wid = lax.axis_index("s") * NC + lax.axis_index("c")
    base = wid * pt

    def chunk_body(g, carry):
        off = base + g * C
        pltpu.sync_copy(x_hbm.at[pl.ds(off, C)], x_v)

        def p1(j, carry):
            sl = pl.ds(j * L, L)
            xv = x_v[sl]
            t = (xv - XMIN) / h
            ii = t.astype(jnp.int32)
            u = jnp.minimum(jnp.maximum(t - ii.astype(jnp.float32), 0.0),
                            1.0 - eps)
            u2 = u * u
            u3 = u2 * u
            s = 1.0 - u
            w0_v[sl] = s * s * s / 6.0
            w1_v[sl] = (3.0 * u3 - 6.0 * u2 + 4.0) / 6.0
            w2_v[sl] = (-3.0 * u3 + 3.0 * u2 + 3.0 * u + 1.0) / 6.0
            w3_v[sl] = u3 / 6.0
            a0 = jnp.minimum(jnp.maximum(ii, 0), n - 1) * 4
            i0_v[sl] = a0
            i1_v[sl] = a0 + 1
            i2_v[sl] = a0 + 2
            i3_v[sl] = a0 + 3
            return carry

        lax.fori_loop(0, JJ, p1, 0)

        handles = []
        for b in range(G):
            gsl = pl.ds(b * GROUP, GROUP)
            for iv, cv in ((i0_v, c0_v), (i1_v, c1_v),
                           (i2_v, c2_v), (i3_v, c3_v)):
                handles.append(
                    pltpu.async_copy(c4_hbm.at[iv.at[gsl]], cv.at[gsl], sem))
        for hd in handles:
            hd.wait()

        def p2(j, carry):
            sl = pl.ds(j * L, L)
            acc = (w0_v[sl] * c0_v[sl] + w1_v[sl] * c1_v[sl]
                   + w2_v[sl] * c2_v[sl] + w3_v[sl] * c3_v[sl])
            y_v[sl] = acc
            return carry

        lax.fori_loop(0, JJ, p2, 0)
        pltpu.sync_copy(y_v, y_hbm.at[pl.ds(off, C)])
        return carry

    lax.fori_loop(0, pt // C, chunk_body, 0)


def kernel(x, coeff):
    n = coeff.shape[0]
    shape = x.shape
    xf = x.reshape(-1)
    p = xf.shape[0]
    tile_pts = -(-p // (NW * C)) * C
    p_pad = tile_pts * NW
    if p_pad != p:
        xf = jnp.pad(xf, (0, p_pad - p))

    cm1 = jnp.concatenate([coeff[:1], coeff[:-1]])
    cp1 = jnp.concatenate([coeff[1:], coeff[-1:]])
    cp2 = jnp.concatenate([coeff[2:], coeff[-1:], coeff[-1:]])
    c4 = jnp.stack([cm1, coeff, cp1, cp2], axis=1).reshape(-1)

    mesh = plsc.VectorSubcoreMesh(core_axis_name="c", subcore_axis_name="s")
    run = pl.kernel(
        functools.partial(_spline_body, n, tile_pts),
        out_type=jax.ShapeDtypeStruct((p_pad,), jnp.float32),
        mesh=mesh,
        scratch_types=[
            pltpu.VMEM((C,), jnp.float32),
            pltpu.VMEM((C,), jnp.int32),
            pltpu.VMEM((C,), jnp.int32),
            pltpu.VMEM((C,), jnp.int32),
            pltpu.VMEM((C,), jnp.int32),
            pltpu.VMEM((C,), jnp.float32),
            pltpu.VMEM((C,), jnp.float32),
            pltpu.VMEM((C,), jnp.float32),
            pltpu.VMEM((C,), jnp.float32),
            pltpu.VMEM((C,), jnp.float32),
            pltpu.VMEM((C,), jnp.float32),
            pltpu.VMEM((C,), jnp.float32),
            pltpu.VMEM((C,), jnp.float32),
            pltpu.VMEM((C,), jnp.float32),
            pltpu.SemaphoreType.DMA,
        ],
    )
    y = run(xf, c4)
    if p_pad != p:
        y = y[:p]
    return y.reshape(shape)


# speedup vs baseline: 185.5709x; 3.0332x over previous
import functools

import jax
import jax.numpy as jnp
import numpy as np
from jax import lax
from jax.experimental import pallas as pl
from jax.experimental.pallas import tpu as pltpu
from jax.experimental.pallas import tpu_sc as plsc

XMIN = 0.0
XMAX = 1.0

NC = 2
NS = 16
L = 16
NW = NC * NS

C = 2048
GROUP = 128
G = C // GROUP
JJ = C // L


def _spline_body(n, pt, x_hbm, coeff_hbm, y_hbm,
                 x_v, i0_v, i1_v, i2_v, i3_v,
                 w0_v, w1_v, w2_v, w3_v,
                 c0_v, c1_v, c2_v, c3_v, y_v, sem):
    inv_h = (n - 1) / (XMAX - XMIN)
    sixth = 1.0 / 6.0
    eps = float(np.finfo(np.float32).eps)
    wid = lax.axis_index("s") * NC + lax.axis_index("c")
    base = wid * pt

    def chunk_body(g, carry):
        off = base + g * C
        pltpu.sync_copy(x_hbm.at[pl.ds(off, C)], x_v)

        def p1(j, carry):
            sl = pl.ds(j * L, L)
            xv = x_v[sl]
            t = (xv - XMIN) * inv_h
            ii = t.astype(jnp.int32)
            u = jnp.minimum(jnp.maximum(t - ii.astype(jnp.float32), 0.0),
                            1.0 - eps)
            u2 = u * u
            u3 = u2 * u
            s = 1.0 - u
            w0_v[sl] = s * s * s * sixth
            w1_v[sl] = (3.0 * u3 - 6.0 * u2 + 4.0) * sixth
            w2_v[sl] = (-3.0 * u3 + 3.0 * u2 + 3.0 * u + 1.0) * sixth
            w3_v[sl] = u3 * sixth
            ii = jnp.minimum(jnp.maximum(ii, 0), n - 1)
            i0_v[sl] = jnp.maximum(ii - 1, 0)
            i1_v[sl] = ii
            i2_v[sl] = jnp.minimum(ii + 1, n - 1)
            i3_v[sl] = jnp.minimum(ii + 2, n - 1)
            return carry

        lax.fori_loop(0, JJ, p1, 0)

        handles = []
        for b in range(G):
            gsl = pl.ds(b * GROUP, GROUP)
            for iv, cv in ((i0_v, c0_v), (i1_v, c1_v),
                           (i2_v, c2_v), (i3_v, c3_v)):
                handles.append(
                    pltpu.async_copy(coeff_hbm.at[iv.at[gsl]], cv.at[gsl],
                                     sem))
        for hd in handles:
            hd.wait()

        def p2(j, carry):
            sl = pl.ds(j * L, L)
            acc = (w0_v[sl] * c0_v[sl] + w1_v[sl] * c1_v[sl]
                   + w2_v[sl] * c2_v[sl] + w3_v[sl] * c3_v[sl])
            y_v[sl] = acc
            return carry

        lax.fori_loop(0, JJ, p2, 0)
        pltpu.sync_copy(y_v, y_hbm.at[pl.ds(off, C)])
        return carry

    lax.fori_loop(0, pt // C, chunk_body, 0)


def kernel(x, coeff):
    n = coeff.shape[0]
    shape = x.shape
    xf = x.reshape(-1)
    p = xf.shape[0]
    tile_pts = -(-p // (NW * C)) * C
    p_pad = tile_pts * NW
    if p_pad != p:
        xf = jnp.pad(xf, (0, p_pad - p))

    mesh = plsc.VectorSubcoreMesh(core_axis_name="c", subcore_axis_name="s")
    run = pl.kernel(
        functools.partial(_spline_body, n, tile_pts),
        out_type=jax.ShapeDtypeStruct((p_pad,), jnp.float32),
        mesh=mesh,
        scratch_types=[
            pltpu.VMEM((C,), jnp.float32),
            pltpu.VMEM((C,), jnp.int32),
            pltpu.VMEM((C,), jnp.int32),
            pltpu.VMEM((C,), jnp.int32),
            pltpu.VMEM((C,), jnp.int32),
            pltpu.VMEM((C,), jnp.float32),
            pltpu.VMEM((C,), jnp.float32),
            pltpu.VMEM((C,), jnp.float32),
            pltpu.VMEM((C,), jnp.float32),
            pltpu.VMEM((C,), jnp.float32),
            pltpu.VMEM((C,), jnp.float32),
            pltpu.VMEM((C,), jnp.float32),
            pltpu.VMEM((C,), jnp.float32),
            pltpu.VMEM((C,), jnp.float32),
            pltpu.SemaphoreType.DMA,
        ],
    )
    y = run(xf, coeff)
    if p_pad != p:
        y = y[:p]
    return y.reshape(shape)


# speedup vs baseline: 218.8319x; 1.1792x over previous
import functools

import jax
import jax.numpy as jnp
import numpy as np
from jax import lax
from jax.experimental import pallas as pl
from jax.experimental.pallas import tpu as pltpu
from jax.experimental.pallas import tpu_sc as plsc

XMIN = 0.0
XMAX = 1.0

NC = 2
NS = 16
L = 16
NW = NC * NS

C = 2048
GROUP = 128
G = C // GROUP
JJ = C // L


def _spline_body(n, pt, x_hbm, coeff_hbm, y_hbm, *refs):
    (x_a, i0_a, i1_a, i2_a, i3_a, w0_a, w1_a, w2_a, w3_a,
     c0_a, c1_a, c2_a, c3_a, y_a,
     x_b, i0_b, i1_b, i2_b, i3_b, w0_b, w1_b, w2_b, w3_b,
     c0_b, c1_b, c2_b, c3_b, y_b, sem_a, sem_b) = refs
    sets = (
        (x_a, (i0_a, i1_a, i2_a, i3_a), (w0_a, w1_a, w2_a, w3_a),
         (c0_a, c1_a, c2_a, c3_a), y_a, sem_a),
        (x_b, (i0_b, i1_b, i2_b, i3_b), (w0_b, w1_b, w2_b, w3_b),
         (c0_b, c1_b, c2_b, c3_b), y_b, sem_b),
    )
    inv_h = (n - 1) / (XMAX - XMIN)
    sixth = 1.0 / 6.0
    eps = float(np.finfo(np.float32).eps)
    wid = lax.axis_index("s") * NC + lax.axis_index("c")
    base = wid * pt
    nch = pt // C

    def stage1(g, s):
        x_v, iv4, wv4, cv4, _, sem = sets[s]
        off = base + g * C
        pltpu.sync_copy(x_hbm.at[pl.ds(off, C)], x_v)

        def p1(j, carry):
            sl = pl.ds(j * L, L)
            xv = x_v[sl]
            t = (xv - XMIN) * inv_h
            ii = t.astype(jnp.int32)
            u = jnp.minimum(jnp.maximum(t - ii.astype(jnp.float32), 0.0),
                            1.0 - eps)
            u2 = u * u
            u3 = u2 * u
            sm = 1.0 - u
            wv4[0][sl] = sm * sm * sm * sixth
            wv4[1][sl] = (3.0 * u3 - 6.0 * u2 + 4.0) * sixth
            wv4[2][sl] = (-3.0 * u3 + 3.0 * u2 + 3.0 * u + 1.0) * sixth
            wv4[3][sl] = u3 * sixth
            ii = jnp.minimum(jnp.maximum(ii, 0), n - 1)
            iv4[0][sl] = jnp.maximum(ii - 1, 0)
            iv4[1][sl] = ii
            iv4[2][sl] = jnp.minimum(ii + 1, n - 1)
            iv4[3][sl] = jnp.minimum(ii + 2, n - 1)
            return carry

        lax.fori_loop(0, JJ, p1, 0)
        for b in range(G):
            gsl = pl.ds(b * GROUP, GROUP)
            for iv, cv in zip(iv4, cv4):
                pltpu.async_copy(coeff_hbm.at[iv.at[gsl]], cv.at[gsl], sem)

    def stage2(g, s):
        _, _, wv4, cv4, y_v, sem = sets[s]
        off = base + g * C
        for cv in cv4:
            pltpu.make_async_copy(coeff_hbm.at[pl.ds(0, C)], cv, sem).wait()

        def p2(j, carry):
            sl = pl.ds(j * L, L)
            y_v[sl] = (wv4[0][sl] * cv4[0][sl] + wv4[1][sl] * cv4[1][sl]
                       + wv4[2][sl] * cv4[2][sl] + wv4[3][sl] * cv4[3][sl])
            return carry

        lax.fori_loop(0, JJ, p2, 0)
        pltpu.sync_copy(y_v, y_hbm.at[pl.ds(off, C)])

    stage1(0, 0)

    def pair(g2, carry):
        ga = 2 * g2
        gb = ga + 1
        stage1(gb, 1)
        stage2(ga, 0)

        @pl.when(g2 + 1 < nch // 2)
        def _():
            stage1(ga + 2, 0)

        stage2(gb, 1)
        return carry

    lax.fori_loop(0, nch // 2, pair, 0)


def kernel(x, coeff):
    n = coeff.shape[0]
    shape = x.shape
    xf = x.reshape(-1)
    p = xf.shape[0]
    per_tile = 2 * C
    tile_pts = -(-p // (NW * per_tile)) * per_tile
    p_pad = tile_pts * NW
    if p_pad != p:
        xf = jnp.pad(xf, (0, p_pad - p))

    buf_set = [
        pltpu.VMEM((C,), jnp.float32),
        pltpu.VMEM((C,), jnp.int32),
        pltpu.VMEM((C,), jnp.int32),
        pltpu.VMEM((C,), jnp.int32),
        pltpu.VMEM((C,), jnp.int32),
        pltpu.VMEM((C,), jnp.float32),
        pltpu.VMEM((C,), jnp.float32),
        pltpu.VMEM((C,), jnp.float32),
        pltpu.VMEM((C,), jnp.float32),
        pltpu.VMEM((C,), jnp.float32),
        pltpu.VMEM((C,), jnp.float32),
        pltpu.VMEM((C,), jnp.float32),
        pltpu.VMEM((C,), jnp.float32),
        pltpu.VMEM((C,), jnp.float32),
    ]
    mesh = plsc.VectorSubcoreMesh(core_axis_name="c", subcore_axis_name="s")
    run = pl.kernel(
        functools.partial(_spline_body, n, tile_pts),
        out_type=jax.ShapeDtypeStruct((p_pad,), jnp.float32),
        mesh=mesh,
        scratch_types=buf_set + buf_set
        + [pltpu.SemaphoreType.DMA, pltpu.SemaphoreType.DMA],
    )
    y = run(xf, coeff)
    if p_pad != p:
        y = y[:p]
    return y.reshape(shape)


# speedup vs baseline: 309.9913x; 1.4166x over previous
import functools

import jax
import jax.numpy as jnp
import numpy as np
from jax import lax
from jax.experimental import pallas as pl
from jax.experimental.pallas import tpu as pltpu
from jax.experimental.pallas import tpu_sc as plsc

XMIN = 0.0
XMAX = 1.0

NC = 2
NS = 16
L = 16
NW = NC * NS

D = 16
C = 1024
GROUP = 128
G = C // GROUP
JJ = C // L


def _spline_body(n, nb, pt, x_hbm, tbl_hbm, y_hbm, *refs):
    (x_a, qa_a, qb_a, pos_a, w0_a, w1_a, w2_a, w3_a, rows_a, y_a,
     x_b, qa_b, qb_b, pos_b, w0_b, w1_b, w2_b, w3_b, rows_b, y_b,
     sem_a, sem_b) = refs
    sets = (
        (x_a, qa_a, qb_a, pos_a, (w0_a, w1_a, w2_a, w3_a), rows_a, y_a,
         sem_a),
        (x_b, qa_b, qb_b, pos_b, (w0_b, w1_b, w2_b, w3_b), rows_b, y_b,
         sem_b),
    )
    inv_h = (n - 1) / (XMAX - XMIN)
    sixth = 1.0 / 6.0
    eps = float(np.finfo(np.float32).eps)
    wid = lax.axis_index("s") * NC + lax.axis_index("c")
    base = wid * pt
    nch = pt // C

    def stage1(g, s):
        x_v, qa_v, qb_v, pos_v, wv4, rows_v, _, sem = sets[s]
        off = base + g * C
        pltpu.sync_copy(x_hbm.at[pl.ds(off, C)], x_v)

        def p1(j, carry):
            sl = pl.ds(j * L, L)
            xv = x_v[sl]
            t = (xv - XMIN) * inv_h
            ii = t.astype(jnp.int32)
            u = jnp.minimum(jnp.maximum(t - ii.astype(jnp.float32), 0.0),
                            1.0 - eps)
            u2 = u * u
            u3 = u2 * u
            sm = 1.0 - u
            wv4[0][sl] = sm * sm * sm * sixth
            wv4[1][sl] = (3.0 * u3 - 6.0 * u2 + 4.0) * sixth
            wv4[2][sl] = (-3.0 * u3 + 3.0 * u2 + 3.0 * u + 1.0) * sixth
            wv4[3][sl] = u3 * sixth
            ii = jnp.minimum(jnp.maximum(ii, 0), n - 1)
            q = jnp.minimum(jnp.maximum((ii - 1) >> 4, 0), nb - 2)
            qd = q * D
            cl0 = jnp.maximum(ii - 1, 0)
            cl3 = jnp.minimum(ii + 2, n - 1)
            p0 = jnp.clip(cl0 - qd, 0, 2 * D - 1)
            p1_ = jnp.clip(ii - qd, 0, 2 * D - 1)
            p2_ = jnp.clip(ii + 1 - qd, 0, 2 * D - 1)
            p3 = jnp.clip(cl3 - qd, 0, 2 * D - 1)
            pos_v[sl] = p0 | (p1_ << 5) | (p2_ << 10) | (p3 << 15)
            qa_v[sl] = q
            qb_v[sl] = q + 1
            return carry

        lax.fori_loop(0, JJ, p1, 0)
        for b in range(G):
            gsl = pl.ds(b * GROUP, GROUP)
            pltpu.async_copy(tbl_hbm.at[qa_v.at[gsl]], rows_v.at[gsl], sem)
            pltpu.async_copy(tbl_hbm.at[qb_v.at[gsl]],
                             rows_v.at[pl.ds(C + b * GROUP, GROUP)], sem)

    def stage2(g, s):
        _, _, _, pos_v, wv4, rows_v, y_v, sem = sets[s]
        off = base + g * C
        pltpu.make_async_copy(tbl_hbm.at[pl.ds(0, C)],
                              rows_v.at[pl.ds(0, C)], sem).wait()
        pltpu.make_async_copy(tbl_hbm.at[pl.ds(0, C)],
                              rows_v.at[pl.ds(C, C)], sem).wait()

        def p2(j, carry):
            sl = pl.ds(j * L, L)
            pv = j * L + lax.iota(jnp.int32, L)
            packed = pos_v[sl]
            acc = None
            for k in range(4):
                pk = (packed >> (5 * k)) & 31
                row = pv + (pk >> 4) * C
                col = pk & (D - 1)
                ck = plsc.load_gather(rows_v, [row, col])
                wk = wv4[k][sl]
                acc = wk * ck if acc is None else acc + wk * ck
            y_v[sl] = acc
            return carry

        lax.fori_loop(0, JJ, p2, 0)
        pltpu.sync_copy(y_v, y_hbm.at[pl.ds(off, C)])

    stage1(0, 0)

    def pair(g2, carry):
        ga = 2 * g2
        gb = ga + 1
        stage1(gb, 1)
        stage2(ga, 0)

        @pl.when(g2 + 1 < nch // 2)
        def _():
            stage1(ga + 2, 0)

        stage2(gb, 1)
        return carry

    lax.fori_loop(0, nch // 2, pair, 0)


def kernel(x, coeff):
    n = coeff.shape[0]
    shape = x.shape
    xf = x.reshape(-1)
    p = xf.shape[0]
    per_tile = 2 * C
    tile_pts = -(-p // (NW * per_tile)) * per_tile
    p_pad = tile_pts * NW
    if p_pad != p:
        xf = jnp.pad(xf, (0, p_pad - p))

    nb = -(-n // D)
    cr = coeff
    if nb * D != n:
        cr = jnp.pad(coeff, (0, nb * D - n), mode="edge")
    tbl = jax.lax.optimization_barrier(cr.reshape(nb, D))

    buf_set = [
        pltpu.VMEM((C,), jnp.float32),
        pltpu.VMEM((C,), jnp.int32),
        pltpu.VMEM((C,), jnp.int32),
        pltpu.VMEM((C,), jnp.int32),
        pltpu.VMEM((C,), jnp.float32),
        pltpu.VMEM((C,), jnp.float32),
        pltpu.VMEM((C,), jnp.float32),
        pltpu.VMEM((C,), jnp.float32),
        pltpu.VMEM((2 * C, D), jnp.float32),
        pltpu.VMEM((C,), jnp.float32),
    ]
    mesh = plsc.VectorSubcoreMesh(core_axis_name="c", subcore_axis_name="s")
    run = pl.kernel(
        functools.partial(_spline_body, n, nb, tile_pts),
        out_type=jax.ShapeDtypeStruct((p_pad,), jnp.float32),
        mesh=mesh,
        compiler_params=pltpu.CompilerParams(
            use_tc_tiling_on_sc=False, needs_layout_passes=False),
        scratch_types=buf_set + buf_set
        + [pltpu.SemaphoreType.DMA, pltpu.SemaphoreType.DMA],
    )
    y = run(xf, tbl)
    if p_pad != p:
        y = y[:p]
    return y.reshape(shape)
